# speedup vs baseline: 1.4126x; 1.4126x over previous
import functools

import jax
import jax.numpy as jnp
from jax.experimental import pallas as pl
from jax.experimental.pallas import tpu as pltpu

S, HS = 2048, 1024
E, TOPK = 8, 2
DFF, DSH = 1408, 2816


def _router_body(x_ref, gw_ref, logits_ref, w_ref, eids_ref):
    x = x_ref[...]
    logits = jnp.dot(x, gw_ref[...], preferred_element_type=jnp.float32)
    logits_ref[...] = logits
    p = jax.nn.softmax(logits, axis=-1)
    iota = jax.lax.broadcasted_iota(jnp.int32, p.shape, 1)
    m1 = jnp.max(p, axis=-1, keepdims=True)
    e1 = jnp.min(jnp.where(p == m1, iota, E), axis=-1, keepdims=True)
    p2 = jnp.where(iota == e1, -jnp.inf, p)
    m2 = jnp.max(p2, axis=-1, keepdims=True)
    e2 = jnp.min(jnp.where(p2 == m2, iota, E), axis=-1, keepdims=True)
    tot = m1 + m2
    w_ref[...] = jnp.concatenate([m1 / tot, m2 / tot], axis=-1)
    eids_ref[...] = jnp.concatenate([e1, e2], axis=-1)


def _router(x, gate_w):
    return pl.pallas_call(
        _router_body,
        out_shape=(
            jax.ShapeDtypeStruct((S, E), jnp.float32),
            jax.ShapeDtypeStruct((S, TOPK), jnp.float32),
            jax.ShapeDtypeStruct((S, TOPK), jnp.int32),
        ),
    )(x, gate_w)


def _experts_body(x_ref, wg_ref, wu_ref, wd_ref, w_ref, eids_ref, out_ref):
    e = pl.program_id(1)

    @pl.when(e == 0)
    def _():
        out_ref[...] = jnp.zeros_like(out_ref)

    x = x_ref[...]
    g = jnp.dot(x, wg_ref[0], preferred_element_type=jnp.float32)
    u = jnp.dot(x, wu_ref[0], preferred_element_type=jnp.float32)
    h = (g * jax.nn.sigmoid(g)) * u
    o = jnp.dot(h, wd_ref[0], preferred_element_type=jnp.float32)
    wt = jnp.sum(jnp.where(eids_ref[...] == e, w_ref[...], 0.0), axis=-1,
                 keepdims=True)
    out_ref[...] += wt * o


def _experts(x, gate_proj_w, up_proj_w, down_proj_w, w, eids, bt=512):
    nt = S // bt
    return pl.pallas_call(
        _experts_body,
        grid=(nt, E),
        in_specs=[
            pl.BlockSpec((bt, HS), lambda t, e: (t, 0)),
            pl.BlockSpec((1, HS, DFF), lambda t, e: (e, 0, 0)),
            pl.BlockSpec((1, HS, DFF), lambda t, e: (e, 0, 0)),
            pl.BlockSpec((1, DFF, HS), lambda t, e: (e, 0, 0)),
            pl.BlockSpec((bt, TOPK), lambda t, e: (t, 0)),
            pl.BlockSpec((bt, TOPK), lambda t, e: (t, 0)),
        ],
        out_specs=pl.BlockSpec((bt, HS), lambda t, e: (t, 0)),
        out_shape=jax.ShapeDtypeStruct((S, HS), jnp.float32),
    )(x, gate_proj_w, up_proj_w, down_proj_w, w, eids)


def _shared_body(x_ref, wg_ref, wu_ref, wd_ref, sg_ref, out_ref):
    d = pl.program_id(1)
    nd = pl.num_programs(1)

    @pl.when(d == 0)
    def _():
        out_ref[...] = jnp.zeros_like(out_ref)

    x = x_ref[...]
    g = jnp.dot(x, wg_ref[...], preferred_element_type=jnp.float32)
    u = jnp.dot(x, wu_ref[...], preferred_element_type=jnp.float32)
    h = (g * jax.nn.sigmoid(g)) * u
    out_ref[...] += jnp.dot(h, wd_ref[...], preferred_element_type=jnp.float32)

    @pl.when(d == nd - 1)
    def _():
        gate = jax.nn.sigmoid(
            jnp.dot(x, sg_ref[...], preferred_element_type=jnp.float32))
        out_ref[...] *= gate


def _shared(x, wg, wu, wd, sgw, bt=512, bd=1408):
    nt, nd = S // bt, DSH // bd
    return pl.pallas_call(
        _shared_body,
        grid=(nt, nd),
        in_specs=[
            pl.BlockSpec((bt, HS), lambda t, d: (t, 0)),
            pl.BlockSpec((HS, bd), lambda t, d: (0, d)),
            pl.BlockSpec((HS, bd), lambda t, d: (0, d)),
            pl.BlockSpec((bd, HS), lambda t, d: (d, 0)),
            pl.BlockSpec((HS, 1), lambda t, d: (0, 0)),
        ],
        out_specs=pl.BlockSpec((bt, HS), lambda t, d: (t, 0)),
        out_shape=jax.ShapeDtypeStruct((S, HS), jnp.float32),
    )(x, wg, wu, wd, sgw)


def kernel(hidden_states, gate_w, gate_proj_w, up_proj_w, down_proj_w,
           shared_gate_proj_w, shared_up_proj_w, shared_down_proj_w,
           shared_expert_gate_w):
    x = hidden_states.reshape(S, HS)
    logits, w, eids = _router(x, gate_w)
    moe = _experts(x, gate_proj_w, up_proj_w, down_proj_w, w, eids)
    sh = _shared(x, shared_gate_proj_w, shared_up_proj_w, shared_down_proj_w,
                 shared_expert_gate_w)
    out = (moe + sh).reshape(1, S, HS)
    return (out, logits.reshape(1, S, E))


# speedup vs baseline: 1.4169x; 1.0030x over previous
import functools

import jax
import jax.numpy as jnp
from jax.experimental import pallas as pl
from jax.experimental.pallas import tpu as pltpu

S, HS = 2048, 1024
E, TOPK = 8, 2
DFF, DSH = 1408, 2816


def _router_body(x_ref, gw_ref, logits_ref, w_ref, eids_ref):
    x = x_ref[...]
    logits = jnp.dot(x, gw_ref[...], preferred_element_type=jnp.float32)
    logits_ref[...] = logits
    p = jax.nn.softmax(logits, axis=-1)
    iota = jax.lax.broadcasted_iota(jnp.int32, p.shape, 1)
    m1 = jnp.max(p, axis=-1, keepdims=True)
    e1 = jnp.min(jnp.where(p == m1, iota, E), axis=-1, keepdims=True)
    p2 = jnp.where(iota == e1, -jnp.inf, p)
    m2 = jnp.max(p2, axis=-1, keepdims=True)
    e2 = jnp.min(jnp.where(p2 == m2, iota, E), axis=-1, keepdims=True)
    tot = m1 + m2
    w_ref[...] = jnp.concatenate([m1 / tot, m2 / tot], axis=-1)
    eids_ref[...] = jnp.concatenate([e1, e2], axis=-1)


def _router(x, gate_w):
    return pl.pallas_call(
        _router_body,
        out_shape=(
            jax.ShapeDtypeStruct((S, E), jnp.float32),
            jax.ShapeDtypeStruct((S, TOPK), jnp.float32),
            jax.ShapeDtypeStruct((S, TOPK), jnp.int32),
        ),
    )(x, gate_w)


def _experts_body(x_ref, wg_ref, wu_ref, wd_ref, w_ref, eids_ref, out_ref):
    e = pl.program_id(1)

    @pl.when(e == 0)
    def _():
        out_ref[...] = jnp.zeros_like(out_ref)

    x = x_ref[...].astype(jnp.bfloat16)
    g = jnp.dot(x, wg_ref[0].astype(jnp.bfloat16),
                preferred_element_type=jnp.float32)
    u = jnp.dot(x, wu_ref[0].astype(jnp.bfloat16),
                preferred_element_type=jnp.float32)
    h = ((g * jax.nn.sigmoid(g)) * u).astype(jnp.bfloat16)
    o = jnp.dot(h, wd_ref[0].astype(jnp.bfloat16),
                preferred_element_type=jnp.float32)
    wt = jnp.sum(jnp.where(eids_ref[...] == e, w_ref[...], 0.0), axis=-1,
                 keepdims=True)
    out_ref[...] += wt * o


def _experts(x, gate_proj_w, up_proj_w, down_proj_w, w, eids, bt=512):
    nt = S // bt
    return pl.pallas_call(
        _experts_body,
        grid=(nt, E),
        in_specs=[
            pl.BlockSpec((bt, HS), lambda t, e: (t, 0)),
            pl.BlockSpec((1, HS, DFF), lambda t, e: (e, 0, 0)),
            pl.BlockSpec((1, HS, DFF), lambda t, e: (e, 0, 0)),
            pl.BlockSpec((1, DFF, HS), lambda t, e: (e, 0, 0)),
            pl.BlockSpec((bt, TOPK), lambda t, e: (t, 0)),
            pl.BlockSpec((bt, TOPK), lambda t, e: (t, 0)),
        ],
        out_specs=pl.BlockSpec((bt, HS), lambda t, e: (t, 0)),
        out_shape=jax.ShapeDtypeStruct((S, HS), jnp.float32),
    )(x, gate_proj_w, up_proj_w, down_proj_w, w, eids)


def _shared_body(x_ref, wg_ref, wu_ref, wd_ref, sg_ref, out_ref):
    d = pl.program_id(1)
    nd = pl.num_programs(1)

    @pl.when(d == 0)
    def _():
        out_ref[...] = jnp.zeros_like(out_ref)

    x = x_ref[...].astype(jnp.bfloat16)
    g = jnp.dot(x, wg_ref[...].astype(jnp.bfloat16),
                preferred_element_type=jnp.float32)
    u = jnp.dot(x, wu_ref[...].astype(jnp.bfloat16),
                preferred_element_type=jnp.float32)
    h = ((g * jax.nn.sigmoid(g)) * u).astype(jnp.bfloat16)
    out_ref[...] += jnp.dot(h, wd_ref[...].astype(jnp.bfloat16),
                            preferred_element_type=jnp.float32)

    @pl.when(d == nd - 1)
    def _():
        gate = jax.nn.sigmoid(
            jnp.dot(x_ref[...], sg_ref[...],
                    preferred_element_type=jnp.float32))
        out_ref[...] *= gate


def _shared(x, wg, wu, wd, sgw, bt=512, bd=1408):
    nt, nd = S // bt, DSH // bd
    return pl.pallas_call(
        _shared_body,
        grid=(nt, nd),
        in_specs=[
            pl.BlockSpec((bt, HS), lambda t, d: (t, 0)),
            pl.BlockSpec((HS, bd), lambda t, d: (0, d)),
            pl.BlockSpec((HS, bd), lambda t, d: (0, d)),
            pl.BlockSpec((bd, HS), lambda t, d: (d, 0)),
            pl.BlockSpec((HS, 1), lambda t, d: (0, 0)),
        ],
        out_specs=pl.BlockSpec((bt, HS), lambda t, d: (t, 0)),
        out_shape=jax.ShapeDtypeStruct((S, HS), jnp.float32),
    )(x, wg, wu, wd, sgw)


def kernel(hidden_states, gate_w, gate_proj_w, up_proj_w, down_proj_w,
           shared_gate_proj_w, shared_up_proj_w, shared_down_proj_w,
           shared_expert_gate_w):
    x = hidden_states.reshape(S, HS)
    logits, w, eids = _router(x, gate_w)
    moe = _experts(x, gate_proj_w, up_proj_w, down_proj_w, w, eids)
    sh = _shared(x, shared_gate_proj_w, shared_up_proj_w, shared_down_proj_w,
                 shared_expert_gate_w)
    out = (moe + sh).reshape(1, S, HS)
    return (out, logits.reshape(1, S, E))
